# baseline (speedup 1.0000x reference)
import jax
import jax.numpy as jnp
from jax import lax
from jax.experimental import pallas as pl
from jax.experimental.pallas import tpu as pltpu
from jax.experimental.pallas import tpu_sc as plsc

B = 1024
S = 200
D = 64
N = B * S
NW = 32
PER_W = N // NW
G = 2
C = G * 128
NCH = PER_W // C
NBUF = 2
NIN = 4
L = 16
NCOMB = 2 * 3 * S


def _body(seq, lbl, tok, comb, out, idx_v, lbl_v, comb_v, bufA,
          in_s, ga_s, out_s):
    wid = lax.axis_index("s") * 2 + lax.axis_index("c")
    rbase = wid * (PER_W // 128)
    iota = lax.iota(jnp.int32, L)

    pltpu.sync_copy(comb, comb_v)

    def fire_in(k):
        s = lax.rem(k, NIN)
        r = rbase + k * G
        pltpu.async_copy(seq.at[pl.ds(r, G)], idx_v.at[s], in_s.at[s])
        pltpu.async_copy(lbl.at[pl.ds(r, G)], lbl_v.at[s], in_s.at[s])

    def wait_in(k):
        s = lax.rem(k, NIN)
        pltpu.make_async_copy(seq.at[pl.ds(0, G)], idx_v.at[s], in_s.at[s]).wait()
        pltpu.make_async_copy(lbl.at[pl.ds(0, G)], lbl_v.at[s], in_s.at[s]).wait()

    def fire_gather(k, b):
        s = lax.rem(k, NIN)
        for i in range(G):
            pltpu.async_copy(
                tok.at[idx_v.at[s, i]],
                bufA.at[pl.ds((b * G + i) * 128, 128)],
                ga_s.at[b],
            )

    def wait_gather(b):
        for i in range(G):
            pltpu.make_async_copy(
                tok.at[idx_v.at[0, 0]],
                bufA.at[pl.ds(i * 128, 128)],
                ga_s.at[b],
            ).wait()

    def add_chunk(k, b):
        s = lax.rem(k, NIN)
        for i in range(G):
            rowg = (rbase + k * G + i) * 128

            @pl.loop(0, 128 // L)
            def _(gg):
                ids = idx_v[s, i, pl.ds(gg * L, L)]
                lbs = lbl_v[s, i, pl.ds(gg * L, L)]
                pos = lax.rem(rowg + gg * L + iota, S)
                cvec = lbs * S + pos + jnp.where(ids == 0, 3 * S, 0)
                rowv = (b * G + i) * 128 + gg * L + iota
                for c in range(D):
                    colv = jnp.full((L,), c, jnp.int32)
                    bv = plsc.load_gather(comb_v, [cvec, colv])
                    plsc.addupdate_scatter(bufA, [rowv, colv], bv)

    def fire_out(k, b):
        pltpu.async_copy(
            bufA.at[pl.ds(b * C, C)],
            out.at[pl.ds((rbase + k * G) * 128, C)],
            out_s.at[b],
        )

    def wait_out(b):
        pltpu.make_async_copy(
            bufA.at[pl.ds(0, C)], out.at[pl.ds(0, C)], out_s.at[b]
        ).wait()

    fire_in(0)
    fire_in(1)
    fire_in(2)
    wait_in(0)
    fire_gather(0, 0)

    def step(k, _):
        b = lax.rem(k, NBUF)
        p = 1 - b
        wait_in(k)

        @pl.when(k >= NBUF)
        def _():
            wait_out(b)

        fire_gather(k, b)

        @pl.when(k + 2 < NCH)
        def _():
            fire_in(k + 2)

        wait_gather(p)
        add_chunk(k - 1, p)
        fire_out(k - 1, p)
        return 0

    lax.fori_loop(1, NCH, step, 0)

    bl = (NCH - 1) % NBUF
    wait_gather(bl)
    add_chunk(NCH - 1, bl)
    fire_out(NCH - 1, bl)
    wait_out(1 - bl)
    wait_out(bl)


def kernel(sequence, segment_label, token_table, segment_table, pe):
    seq = sequence.reshape(N // 128, 128).astype(jnp.int32)
    lbl = segment_label.reshape(N // 128, 128).astype(jnp.int32)

    seg0 = segment_table.at[0].set(0.0)
    base_tab = (seg0[:, None, :] + pe[0][None, :, :]).reshape(3 * S, D)
    comb = jnp.concatenate([base_tab, base_tab - token_table[0][None, :]], axis=0)

    run = pl.kernel(
        _body,
        out_type=jax.ShapeDtypeStruct((N, D), jnp.float32),
        mesh=plsc.VectorSubcoreMesh(core_axis_name="c", subcore_axis_name="s"),
        compiler_params=pltpu.CompilerParams(
            use_tc_tiling_on_sc=False, needs_layout_passes=False
        ),
        scratch_types=[
            pltpu.VMEM((NIN, G, 128), jnp.int32),
            pltpu.VMEM((NIN, G, 128), jnp.int32),
            pltpu.VMEM((NCOMB, D), jnp.float32),
            pltpu.VMEM((NBUF * C, D), jnp.float32),
            pltpu.SemaphoreType.DMA((NIN,)),
            pltpu.SemaphoreType.DMA((NBUF,)),
            pltpu.SemaphoreType.DMA((NBUF,)),
        ],
    )
    out = run(seq, lbl, token_table, comb)
    return out.reshape(B, S, D)


# speedup vs baseline: 1.4665x; 1.4665x over previous
import jax
import jax.numpy as jnp
from jax import lax
from jax.experimental import pallas as pl
from jax.experimental.pallas import tpu as pltpu
from jax.experimental.pallas import tpu_sc as plsc

B = 1024
S = 200
D = 64
N = B * S
NW = 32
PER_W = N // NW
G = 2
C = G * 128
NCH = PER_W // C
NBUF = 2
NIN = 4
L = 16
NCOMB = 2 * 3 * S


def _body(seq, lbl, tok, comb, out,
          idx_v, lbl_v, cidx_v, comb_v, rowsA,
          in_s, ga_s, out_s):
    wid = lax.axis_index("s") * 2 + lax.axis_index("c")
    rbase = wid * (PER_W // 128)
    iota = lax.iota(jnp.int32, L)

    pltpu.sync_copy(comb, comb_v)

    def fire_in(k):
        t = lax.rem(k, NIN)
        r = rbase + k * G
        pltpu.async_copy(seq.at[pl.ds(r, G)], idx_v.at[t], in_s.at[t])
        pltpu.async_copy(lbl.at[pl.ds(r, G)], lbl_v.at[t], in_s.at[t])

    def wait_in(k):
        t = lax.rem(k, NIN)
        pltpu.make_async_copy(seq.at[pl.ds(0, G)], idx_v.at[t], in_s.at[t]).wait()
        pltpu.make_async_copy(lbl.at[pl.ds(0, G)], lbl_v.at[t], in_s.at[t]).wait()

    def fire_gather(k, b):
        t = lax.rem(k, NIN)
        for i in range(G):
            pltpu.async_copy(
                tok.at[idx_v.at[t, i]],
                rowsA.at[b, pl.ds(i * 128, 128)],
                ga_s.at[b],
            )
        rowg = (rbase + k * G) * 128
        for i in range(G):
            for gg in range(128 // L):
                ids = idx_v[t, i, pl.ds(gg * L, L)]
                lbs = lbl_v[t, i, pl.ds(gg * L, L)]
                pos = lax.rem(rowg + i * 128 + gg * L + iota, S)
                cidx_v[t, pl.ds(i * 128 + gg * L, L)] = (
                    lbs * S + pos + jnp.where(ids == 0, 3 * S, 0)
                )

    def wait_gather(b):
        for i in range(G):
            pltpu.make_async_copy(
                tok.at[idx_v.at[0, 0]],
                rowsA.at[b, pl.ds(i * 128, 128)],
                ga_s.at[b],
            ).wait()

    def add_chunk(k, b):
        t = lax.rem(k, NIN)

        @pl.loop(0, C // L)
        def _(g):
            cv = cidx_v[t, pl.ds(g * L, L)]
            for l in range(L):
                cid = cv[l]
                r = g * L + l
                for j in range(D // L):
                    rowsA[b, r, pl.ds(j * L, L)] = (
                        rowsA[b, r, pl.ds(j * L, L)]
                        + comb_v[cid, pl.ds(j * L, L)]
                    )

    def fire_out(k, b):
        pltpu.async_copy(
            rowsA.at[b],
            out.at[pl.ds((rbase + k * G) * 128, C)],
            out_s.at[b],
        )

    def wait_out(b):
        pltpu.make_async_copy(
            rowsA.at[b], out.at[pl.ds(0, C)], out_s.at[b]
        ).wait()

    fire_in(0)
    fire_in(1)
    wait_in(0)
    fire_gather(0, 0)
    fire_in(2)

    def step(k, _):
        b = lax.rem(k, NBUF)
        p = 1 - b
        wait_in(k)

        @pl.when(k >= NBUF)
        def _():
            wait_out(b)

        fire_gather(k, b)

        @pl.when(k + 2 < NCH)
        def _():
            fire_in(k + 2)

        wait_gather(p)
        add_chunk(k - 1, p)
        fire_out(k - 1, p)
        return 0

    lax.fori_loop(1, NCH, step, 0)

    bl = (NCH - 1) % NBUF
    wait_gather(bl)
    add_chunk(NCH - 1, bl)
    fire_out(NCH - 1, bl)
    wait_out(1 - bl)
    wait_out(bl)


def kernel(sequence, segment_label, token_table, segment_table, pe):
    seq = sequence.reshape(N // 128, 128).astype(jnp.int32)
    lbl = segment_label.reshape(N // 128, 128).astype(jnp.int32)

    seg0 = segment_table.at[0].set(0.0)
    base_tab = (seg0[:, None, :] + pe[0][None, :, :]).reshape(3 * S, D)
    comb = jnp.concatenate([base_tab, base_tab - token_table[0][None, :]], axis=0)

    run = pl.kernel(
        _body,
        out_type=jax.ShapeDtypeStruct((N, D), jnp.float32),
        mesh=plsc.VectorSubcoreMesh(core_axis_name="c", subcore_axis_name="s"),
        compiler_params=pltpu.CompilerParams(use_tc_tiling_on_sc=False),
        scratch_types=[
            pltpu.VMEM((NIN, G, 128), jnp.int32),
            pltpu.VMEM((NIN, G, 128), jnp.int32),
            pltpu.VMEM((NIN, C), jnp.int32),
            pltpu.VMEM((NCOMB, D), jnp.float32),
            pltpu.VMEM((NBUF, C, D), jnp.float32),
            pltpu.SemaphoreType.DMA((NIN,)),
            pltpu.SemaphoreType.DMA((NBUF,)),
            pltpu.SemaphoreType.DMA((NBUF,)),
        ],
    )
    out = run(seq, lbl, token_table, comb)
    return out.reshape(B, S, D)


# speedup vs baseline: 1.4776x; 1.0076x over previous
import jax
import jax.numpy as jnp
from jax import lax
from jax.experimental import pallas as pl
from jax.experimental.pallas import tpu as pltpu
from jax.experimental.pallas import tpu_sc as plsc

B = 1024
S = 200
D = 64
N = B * S
NW = 32
PER_W = N // NW
G = 2
C = G * 128
NCH = PER_W // C
NBUF = 2
NIN = 4
L = 16
NCOMB = 2 * 3 * S


def _body(seq, lbl, tok, comb, out,
          idx_v, lbl_v, cidx_v, comb_v, rowsA, rowsB,
          in_s, b_s, ga_s, out_s):
    wid = lax.axis_index("s") * 2 + lax.axis_index("c")
    rbase = wid * (PER_W // 128)
    iota = lax.iota(jnp.int32, L)

    @pl.when(lax.axis_index("s") == 0)
    def _():
        pltpu.sync_copy(comb, comb_v)
    plsc.subcore_barrier()

    def fire_in(k):
        t = lax.rem(k, NIN)
        r = rbase + k * G
        pltpu.async_copy(seq.at[pl.ds(r, G)], idx_v.at[t], in_s.at[t])
        pltpu.async_copy(lbl.at[pl.ds(r, G)], lbl_v.at[t], in_s.at[t])

    def wait_in(k):
        t = lax.rem(k, NIN)
        pltpu.make_async_copy(seq.at[pl.ds(0, G)], idx_v.at[t], in_s.at[t]).wait()
        pltpu.make_async_copy(lbl.at[pl.ds(0, G)], lbl_v.at[t], in_s.at[t]).wait()

    def fire_gather(k, b):
        t = lax.rem(k, NIN)
        for i in range(G):
            pltpu.async_copy(
                tok.at[idx_v.at[t, i]],
                rowsA.at[b, pl.ds(i * 128, 128)],
                ga_s.at[b],
            )
        rowg = (rbase + k * G) * 128
        for i in range(G):
            for gg in range(128 // L):
                ids = idx_v[t, i, pl.ds(gg * L, L)]
                lbs = lbl_v[t, i, pl.ds(gg * L, L)]
                pos = lax.rem(rowg + i * 128 + gg * L + iota, S)
                cidx_v[t, i, pl.ds(gg * L, L)] = (
                    lbs * S + pos + jnp.where(ids == 0, 3 * S, 0)
                )

    def wait_gather(b):
        for i in range(G):
            pltpu.make_async_copy(
                tok.at[idx_v.at[0, 0]],
                rowsA.at[b, pl.ds(i * 128, 128)],
                ga_s.at[b],
            ).wait()

    def fire_b(k):
        t = lax.rem(k, NIN)
        for i in range(G):
            pltpu.async_copy(
                comb_v.at[cidx_v.at[t, i]],
                rowsB.at[pl.ds(i * 128, 128)],
                b_s,
            )

    def wait_b():
        for i in range(G):
            pltpu.make_async_copy(
                comb_v.at[cidx_v.at[0, 0]],
                rowsB.at[pl.ds(i * 128, 128)],
                b_s,
            ).wait()

    def add_chunk(b):
        @pl.loop(0, C, unroll=2)
        def _(r):
            for j in range(D // L):
                rowsA[b, r, pl.ds(j * L, L)] = (
                    rowsA[b, r, pl.ds(j * L, L)] + rowsB[r, pl.ds(j * L, L)]
                )

    def fire_out(k, b):
        pltpu.async_copy(
            rowsA.at[b],
            out.at[pl.ds((rbase + k * G) * 128, C)],
            out_s.at[b],
        )

    def wait_out(b):
        pltpu.make_async_copy(
            rowsA.at[b], out.at[pl.ds(0, C)], out_s.at[b]
        ).wait()

    fire_in(0)
    fire_in(1)
    wait_in(0)
    fire_gather(0, 0)
    fire_b(0)
    fire_in(2)

    def step(k, _):
        b = lax.rem(k, NBUF)
        p = 1 - b
        wait_in(k)

        @pl.when(k >= NBUF)
        def _():
            wait_out(b)

        fire_gather(k, b)

        @pl.when(k + 2 < NCH)
        def _():
            fire_in(k + 2)

        wait_gather(p)
        wait_b()
        add_chunk(p)
        fire_out(k - 1, p)
        fire_b(k)
        return 0

    lax.fori_loop(1, NCH, step, 0)

    bl = (NCH - 1) % NBUF
    wait_gather(bl)
    wait_b()
    add_chunk(bl)
    fire_out(NCH - 1, bl)
    wait_out(1 - bl)
    wait_out(bl)


def kernel(sequence, segment_label, token_table, segment_table, pe):
    seq = sequence.reshape(N // 128, 128).astype(jnp.int32)
    lbl = segment_label.reshape(N // 128, 128).astype(jnp.int32)

    seg0 = segment_table.at[0].set(0.0)
    base_tab = (seg0[:, None, :] + pe[0][None, :, :]).reshape(3 * S, D)
    comb = jnp.concatenate([base_tab, base_tab - token_table[0][None, :]], axis=0)

    run = pl.kernel(
        _body,
        out_type=jax.ShapeDtypeStruct((N, D), jnp.float32),
        mesh=plsc.VectorSubcoreMesh(core_axis_name="c", subcore_axis_name="s"),
        compiler_params=pltpu.CompilerParams(use_tc_tiling_on_sc=False),
        scratch_types=[
            pltpu.VMEM((NIN, G, 128), jnp.int32),
            pltpu.VMEM((NIN, G, 128), jnp.int32),
            pltpu.VMEM((NIN, G, 128), jnp.int32),
            pltpu.VMEM_SHARED((NCOMB, D), jnp.float32),
            pltpu.VMEM((NBUF, C, D), jnp.float32),
            pltpu.VMEM((C, D), jnp.float32),
            pltpu.SemaphoreType.DMA((NIN,)),
            pltpu.SemaphoreType.DMA,
            pltpu.SemaphoreType.DMA((NBUF,)),
            pltpu.SemaphoreType.DMA((NBUF,)),
        ],
    )
    out = run(seq, lbl, token_table, comb)
    return out.reshape(B, S, D)


# speedup vs baseline: 1.6447x; 1.1131x over previous
import jax
import jax.numpy as jnp
from jax import lax
from jax.experimental import pallas as pl
from jax.experimental.pallas import tpu as pltpu
from jax.experimental.pallas import tpu_sc as plsc

B = 1024
S = 200
D = 64
N = B * S
NW = 32
PER_W = N // NW
G = 2
C = G * 128
NCH = PER_W // C
NBUF = 2
NIN = 4
L = 16
NCOMB = 2 * 3 * S


def _body(seq, lbl, tok, comb, out,
          idx_v, lbl_v, cidx_v, comb_v, rowsA, rowsB,
          in_s, b_s, ga_s, out_s):
    wid = lax.axis_index("s") * 2 + lax.axis_index("c")
    rbase = wid * (PER_W // 128)
    iota = lax.iota(jnp.int32, L)

    @pl.when(lax.axis_index("s") == 0)
    def _():
        pltpu.sync_copy(comb, comb_v)
    plsc.subcore_barrier()

    def fire_in(k):
        t = lax.rem(k, NIN)
        r = rbase + k * G
        pltpu.async_copy(seq.at[pl.ds(r, G)], idx_v.at[t], in_s.at[t])
        pltpu.async_copy(lbl.at[pl.ds(r, G)], lbl_v.at[t], in_s.at[t])

    def wait_in(k):
        t = lax.rem(k, NIN)
        pltpu.make_async_copy(seq.at[pl.ds(0, G)], idx_v.at[t], in_s.at[t]).wait()
        pltpu.make_async_copy(lbl.at[pl.ds(0, G)], lbl_v.at[t], in_s.at[t]).wait()

    def fire_gather(k, b):
        t = lax.rem(k, NIN)
        for i in range(G):
            pltpu.async_copy(
                tok.at[idx_v.at[t, i]],
                rowsA.at[b, pl.ds(i * 128, 128)],
                ga_s.at[b],
            )
        rowg = (rbase + k * G) * 128
        for i in range(G):
            for gg in range(128 // L):
                ids = idx_v[t, i, pl.ds(gg * L, L)]
                lbs = lbl_v[t, i, pl.ds(gg * L, L)]
                pos = lax.rem(rowg + i * 128 + gg * L + iota, S)
                cidx_v[t, i, pl.ds(gg * L, L)] = (
                    lbs * S + pos + jnp.where(ids == 0, 3 * S, 0)
                )

    def wait_gather(b):
        for i in range(G):
            pltpu.make_async_copy(
                tok.at[idx_v.at[0, 0]],
                rowsA.at[b, pl.ds(i * 128, 128)],
                ga_s.at[b],
            ).wait()

    def fire_b(k):
        t = lax.rem(k, NIN)
        for i in range(G):
            pltpu.async_copy(
                comb_v.at[cidx_v.at[t, i]],
                rowsB.at[pl.ds(i * 128, 128)],
                b_s,
            )

    def wait_b():
        for i in range(G):
            pltpu.make_async_copy(
                comb_v.at[cidx_v.at[0, 0]],
                rowsB.at[pl.ds(i * 128, 128)],
                b_s,
            ).wait()

    def add_chunk(b):
        @plsc.parallel_loop(0, C, 1, unroll=4)
        def _(r):
            for j in range(D // L):
                rowsA[b, r, pl.ds(j * L, L)] = (
                    rowsA[b, r, pl.ds(j * L, L)] + rowsB[r, pl.ds(j * L, L)]
                )

    def fire_out(k, b):
        pltpu.async_copy(
            rowsA.at[b],
            out.at[pl.ds((rbase + k * G) * 128, C)],
            out_s.at[b],
        )

    def wait_out(b):
        pltpu.make_async_copy(
            rowsA.at[b], out.at[pl.ds(0, C)], out_s.at[b]
        ).wait()

    fire_in(0)
    fire_in(1)
    wait_in(0)
    fire_gather(0, 0)
    fire_b(0)
    fire_in(2)

    def step(k, _):
        b = lax.rem(k, NBUF)
        p = 1 - b
        wait_in(k)

        @pl.when(k >= NBUF)
        def _():
            wait_out(b)

        fire_gather(k, b)

        @pl.when(k + 2 < NCH)
        def _():
            fire_in(k + 2)

        wait_gather(p)
        wait_b()
        add_chunk(p)
        fire_out(k - 1, p)
        fire_b(k)
        return 0

    lax.fori_loop(1, NCH, step, 0)

    bl = (NCH - 1) % NBUF
    wait_gather(bl)
    wait_b()
    add_chunk(bl)
    fire_out(NCH - 1, bl)
    wait_out(1 - bl)
    wait_out(bl)


def kernel(sequence, segment_label, token_table, segment_table, pe):
    seq = sequence.reshape(N // 128, 128).astype(jnp.int32)
    lbl = segment_label.reshape(N // 128, 128).astype(jnp.int32)

    seg0 = segment_table.at[0].set(0.0)
    base_tab = (seg0[:, None, :] + pe[0][None, :, :]).reshape(3 * S, D)
    comb = jnp.concatenate([base_tab, base_tab - token_table[0][None, :]], axis=0)

    run = pl.kernel(
        _body,
        out_type=jax.ShapeDtypeStruct((N, D), jnp.float32),
        mesh=plsc.VectorSubcoreMesh(core_axis_name="c", subcore_axis_name="s"),
        compiler_params=pltpu.CompilerParams(use_tc_tiling_on_sc=False),
        scratch_types=[
            pltpu.VMEM((NIN, G, 128), jnp.int32),
            pltpu.VMEM((NIN, G, 128), jnp.int32),
            pltpu.VMEM((NIN, G, 128), jnp.int32),
            pltpu.VMEM_SHARED((NCOMB, D), jnp.float32),
            pltpu.VMEM((NBUF, C, D), jnp.float32),
            pltpu.VMEM((C, D), jnp.float32),
            pltpu.SemaphoreType.DMA((NIN,)),
            pltpu.SemaphoreType.DMA,
            pltpu.SemaphoreType.DMA((NBUF,)),
            pltpu.SemaphoreType.DMA((NBUF,)),
        ],
    )
    out = run(seq, lbl, token_table, comb)
    return out.reshape(B, S, D)


# speedup vs baseline: 1.6656x; 1.0127x over previous
import jax
import jax.numpy as jnp
from jax import lax
from jax.experimental import pallas as pl
from jax.experimental.pallas import tpu as pltpu
from jax.experimental.pallas import tpu_sc as plsc

B = 1024
S = 200
D = 64
N = B * S
NW = 32
PER_W = N // NW
G = 2
C = G * 128
NCH = PER_W // C
NBUF = 3
NIN = 4
L = 16
NCOMB = 2 * 3 * S


def _body(seq, lbl, tok, comb, out,
          idx_v, lbl_v, cidx_v, comb_v, rowsA, rowsB,
          in_s, b_s, ga_s, out_s):
    wid = lax.axis_index("s") * 2 + lax.axis_index("c")
    rbase = wid * (PER_W // 128)
    iota = lax.iota(jnp.int32, L)

    @pl.when(lax.axis_index("s") == 0)
    def _():
        pltpu.sync_copy(comb, comb_v)
    plsc.subcore_barrier()

    def fire_in(k):
        t = lax.rem(k, NIN)
        r = rbase + k * G
        pltpu.async_copy(seq.at[pl.ds(r, G)], idx_v.at[t], in_s.at[t])
        pltpu.async_copy(lbl.at[pl.ds(r, G)], lbl_v.at[t], in_s.at[t])

    def wait_in(k):
        t = lax.rem(k, NIN)
        pltpu.make_async_copy(seq.at[pl.ds(0, G)], idx_v.at[t], in_s.at[t]).wait()
        pltpu.make_async_copy(lbl.at[pl.ds(0, G)], lbl_v.at[t], in_s.at[t]).wait()

    def fire_gather(k, b):
        t = lax.rem(k, NIN)
        for i in range(G):
            pltpu.async_copy(
                tok.at[idx_v.at[t, i]],
                rowsA.at[b, pl.ds(i * 128, 128)],
                ga_s.at[b],
            )
        rowg = (rbase + k * G) * 128
        for i in range(G):
            for gg in range(128 // L):
                ids = idx_v[t, i, pl.ds(gg * L, L)]
                lbs = lbl_v[t, i, pl.ds(gg * L, L)]
                pos = lax.rem(rowg + i * 128 + gg * L + iota, S)
                cidx_v[t, i, pl.ds(gg * L, L)] = (
                    lbs * S + pos + jnp.where(ids == 0, 3 * S, 0)
                )

    def wait_gather(b):
        for i in range(G):
            pltpu.make_async_copy(
                tok.at[idx_v.at[0, 0]],
                rowsA.at[b, pl.ds(i * 128, 128)],
                ga_s.at[b],
            ).wait()

    def fire_b(k):
        t = lax.rem(k, NIN)
        q = lax.rem(k, 2)
        for i in range(G):
            pltpu.async_copy(
                comb_v.at[cidx_v.at[t, i]],
                rowsB.at[q, pl.ds(i * 128, 128)],
                b_s.at[q],
            )

    def wait_b(k):
        q = lax.rem(k, 2)
        for i in range(G):
            pltpu.make_async_copy(
                comb_v.at[cidx_v.at[0, 0]],
                rowsB.at[q, pl.ds(i * 128, 128)],
                b_s.at[q],
            ).wait()

    def add_chunk(k, b):
        q = lax.rem(k, 2)

        @plsc.parallel_loop(0, C, 1, unroll=4)
        def _(r):
            for j in range(D // L):
                rowsA[b, r, pl.ds(j * L, L)] = (
                    rowsA[b, r, pl.ds(j * L, L)] + rowsB[q, r, pl.ds(j * L, L)]
                )

    def fire_out(k, b):
        pltpu.async_copy(
            rowsA.at[b],
            out.at[pl.ds((rbase + k * G) * 128, C)],
            out_s.at[b],
        )

    def wait_out(b):
        pltpu.make_async_copy(
            rowsA.at[b], out.at[pl.ds(0, C)], out_s.at[b]
        ).wait()

    fire_in(0)
    fire_in(1)
    wait_in(0)
    fire_gather(0, 0)
    fire_b(0)
    fire_in(2)

    def step(k, _):
        b = lax.rem(k, NBUF)
        p = lax.rem(k - 1, NBUF)
        wait_in(k)

        @pl.when(k >= NBUF)
        def _():
            wait_out(b)

        fire_gather(k, b)
        fire_b(k)

        @pl.when(k + 2 < NCH)
        def _():
            fire_in(k + 2)

        wait_gather(p)
        wait_b(k - 1)
        add_chunk(k - 1, p)
        fire_out(k - 1, p)
        return 0

    lax.fori_loop(1, NCH, step, 0)

    bl = (NCH - 1) % NBUF
    wait_gather(bl)
    wait_b(NCH - 1)
    add_chunk(NCH - 1, bl)
    fire_out(NCH - 1, bl)
    for bb in range(NBUF):
        wait_out(bb)


def kernel(sequence, segment_label, token_table, segment_table, pe):
    seq = sequence.reshape(N // 128, 128).astype(jnp.int32)
    lbl = segment_label.reshape(N // 128, 128).astype(jnp.int32)

    seg0 = segment_table.at[0].set(0.0)
    base_tab = (seg0[:, None, :] + pe[0][None, :, :]).reshape(3 * S, D)
    comb = jnp.concatenate([base_tab, base_tab - token_table[0][None, :]], axis=0)

    run = pl.kernel(
        _body,
        out_type=jax.ShapeDtypeStruct((N, D), jnp.float32),
        mesh=plsc.VectorSubcoreMesh(core_axis_name="c", subcore_axis_name="s"),
        compiler_params=pltpu.CompilerParams(use_tc_tiling_on_sc=False),
        scratch_types=[
            pltpu.VMEM((NIN, G, 128), jnp.int32),
            pltpu.VMEM((NIN, G, 128), jnp.int32),
            pltpu.VMEM((NIN, G, 128), jnp.int32),
            pltpu.VMEM_SHARED((NCOMB, D), jnp.float32),
            pltpu.VMEM((NBUF, C, D), jnp.float32),
            pltpu.VMEM((2, C, D), jnp.float32),
            pltpu.SemaphoreType.DMA((NIN,)),
            pltpu.SemaphoreType.DMA((2,)),
            pltpu.SemaphoreType.DMA((NBUF,)),
            pltpu.SemaphoreType.DMA((NBUF,)),
        ],
    )
    out = run(seq, lbl, token_table, comb)
    return out.reshape(B, S, D)
